# baseline (speedup 1.0000x reference)
import functools

import jax
import jax.numpy as jnp
from jax.experimental import pallas as pl
from jax.experimental.pallas import tpu as pltpu

H = 32
CIN = 3
K1, C1 = 5, 16
O1 = H - K1 + 1
P1 = O1 // 2
K2, C2 = 3, 36
O2 = P1 - K2 + 1
P2 = O2 // 2
XL = H * CIN
S1 = 512
L1 = 256
S2 = 512
L2 = 256
FC1 = 128
LANES = 128
NOUT = 10
NB = 128


def _net_kernel(x_ref, B1_ref, b1_ref, B2_ref, b2_ref, fw1_ref, fb1_ref,
                fw2_ref, fb2_ref, o_ref, *, nb):
    f32 = jnp.float32
    r0 = x_ref[pl.ds(0, 7), :, :].reshape(7 * nb, 4 * XL)
    r1 = x_ref[pl.ds(1, 7), :, :].reshape(7 * nb, 4 * XL)
    lp = jnp.concatenate([r0, r1], axis=1)
    c1 = jnp.dot(lp, B1_ref[...], preferred_element_type=f32)

    b1v = b1_ref[...]
    ye = jnp.maximum(c1[:, 0 * S1:1 * S1], c1[:, 1 * S1:2 * S1])
    ye = jnp.maximum(ye[:, :L1], ye[:, L1:])
    ye = jnp.maximum(ye + b1v, 0.0).astype(jnp.bfloat16)
    yo = jnp.maximum(c1[:, 2 * S1:3 * S1], c1[:, 3 * S1:4 * S1])
    yo = jnp.maximum(yo[:, :L1], yo[:, L1:])
    yo = jnp.maximum(yo + b1v, 0.0).astype(jnp.bfloat16)

    m = 6 * nb
    ye0 = jax.lax.slice(ye, (0, 0), (m, L1))
    ye1 = jax.lax.slice(ye, (nb, 0), (nb + m, L1))
    yo0 = jax.lax.slice(yo, (0, 0), (m, L1))
    yo1 = jax.lax.slice(yo, (nb, 0), (nb + m, L1))
    le = jnp.concatenate([ye0, yo0, ye1], axis=1)
    lo = jnp.concatenate([yo0, ye1, yo1], axis=1)
    l2 = jnp.concatenate([le, lo], axis=0)
    c2 = jnp.dot(l2, B2_ref[...], preferred_element_type=f32)

    y2 = jnp.maximum(c2[:m, :], c2[m:, :])
    y2 = jnp.maximum(y2[:, :L2], y2[:, L2:])
    y2 = jnp.maximum(y2 + b2_ref[...], 0.0)

    feats = jnp.concatenate(
        [jax.lax.slice(y2, (h * nb, 0), ((h + 1) * nb, L2))
         for h in range(P2)], axis=1).astype(jnp.bfloat16)
    h1 = jnp.dot(feats, fw1_ref[...], preferred_element_type=f32)
    h1 = jnp.maximum(h1 + fb1_ref[...], 0.0).astype(jnp.bfloat16)
    z = jnp.dot(h1, fw2_ref[...], preferred_element_type=f32)
    o_ref[...] = jnp.maximum(z + fb2_ref[...], 0.0)[:, :NOUT]


def _toeplitz_selector(size_in, size_out, k):
    w = jnp.arange(size_in)[:, None, None]
    order = jnp.concatenate(
        [jnp.arange(0, size_out, 2), jnp.arange(1, size_out, 2)])
    j = jnp.arange(k)[None, :, None]
    return (w == order[None, None, :] + j).astype(jnp.float32)


def _build_tables(w1, w2):
    w1r = w1.reshape(K1, K1, CIN, C1)
    e1 = _toeplitz_selector(H, O1, K1)
    b1 = jnp.einsum('wjv,ijco->icwvo', e1, w1r)
    b1 = b1.reshape(K1 * XL, 2, P1 * C1)
    b1 = jnp.pad(b1, ((0, 0), (0, 0), (0, L1 - P1 * C1)))
    b1 = b1.reshape(K1 * XL, S1)
    B1 = jnp.concatenate(
        [jnp.pad(b1, ((XL * p, XL * (3 - p)), (0, 0))) for p in range(4)],
        axis=1)
    B1 = B1.reshape(2, 4, CIN, H, 4 * S1)
    B1 = jnp.transpose(B1, (0, 2, 1, 3, 4)).reshape(2 * 4 * XL, 4 * S1)

    w2r = w2.reshape(K2, K2, C1, C2)
    e2 = _toeplitz_selector(P1, O2, K2)
    b2 = jnp.einsum('wjv,ijco->iwcvo', e2, w2r)
    b2 = b2.reshape(K2, P1 * C1, 2, P2 * C2)
    b2 = jnp.pad(b2, ((0, 0), (0, L1 - P1 * C1), (0, 0),
                      (0, L2 - P2 * C2)))
    B2 = b2.reshape(K2 * L1, S2)
    return B1.astype(jnp.bfloat16), B2.astype(jnp.bfloat16)


def kernel(x_nchw, w1, b1, w2, b2, fc1_w, fc1_b, fc2_w, fc2_b):
    n = x_nchw.shape[0]
    nb = min(NB, n)
    n_pad = ((n + nb - 1) // nb) * nb

    x = x_nchw.astype(jnp.bfloat16).reshape(n, CIN, 8, 4, H)
    x = jnp.transpose(x, (2, 0, 1, 3, 4)).reshape(8, n, 4 * XL)
    if n_pad > n:
        x = jnp.pad(x, ((0, 0), (0, n_pad - n), (0, 0)))

    B1, B2 = _build_tables(w1, w2)
    b1t = jnp.pad(jnp.tile(b1, (1, P1)), ((0, 0), (0, L1 - P1 * C1)))
    b2t = jnp.pad(jnp.tile(b2, (1, P2)), ((0, 0), (0, L2 - P2 * C2)))
    fw1 = fc1_w.reshape(P2, P2 * C2, FC1)
    fw1 = jnp.pad(fw1, ((0, 0), (0, L2 - P2 * C2), (0, 0)))
    fw1 = fw1.reshape(P2 * L2, FC1).astype(jnp.bfloat16)
    fw2 = fc2_w.astype(jnp.bfloat16)

    out = pl.pallas_call(
        functools.partial(_net_kernel, nb=nb),
        out_shape=jax.ShapeDtypeStruct((n_pad, NOUT), jnp.float32),
        grid=(n_pad // nb,),
        in_specs=[
            pl.BlockSpec((8, nb, 4 * XL), lambda i: (0, i, 0)),
            pl.BlockSpec((2 * 4 * XL, 4 * S1), lambda i: (0, 0)),
            pl.BlockSpec((1, L1), lambda i: (0, 0)),
            pl.BlockSpec((K2 * L1, S2), lambda i: (0, 0)),
            pl.BlockSpec((1, L2), lambda i: (0, 0)),
            pl.BlockSpec((P2 * L2, FC1), lambda i: (0, 0)),
            pl.BlockSpec((1, FC1), lambda i: (0, 0)),
            pl.BlockSpec((FC1, LANES), lambda i: (0, 0)),
            pl.BlockSpec((1, LANES), lambda i: (0, 0)),
        ],
        out_specs=pl.BlockSpec((nb, NOUT), lambda i: (i, 0)),
        compiler_params=pltpu.CompilerParams(
            dimension_semantics=("parallel",)),
    )(x, B1, b1t, B2, b2t, fw1, fc1_b, fw2, fc2_b)
    return out[:n] if n_pad > n else out


# speedup vs baseline: 1.1996x; 1.1996x over previous
import functools

import jax
import jax.numpy as jnp
from jax.experimental import pallas as pl
from jax.experimental.pallas import tpu as pltpu

H = 32
CIN = 3
K1, C1 = 5, 16
O1 = H - K1 + 1
P1 = O1 // 2
K2, C2 = 3, 36
O2 = P1 - K2 + 1
P2 = O2 // 2
XL = H * CIN
S1 = 512
L1 = 256
S2 = 512
L2 = 256
FC1 = 128
LANES = 128
NOUT = 10
NB = 256


def _net_kernel(x_ref, B1_ref, b1_ref, B2_ref, b2_ref, fw1_ref, fb1_ref,
                fw2_ref, fb2_ref, o_ref, *, nb):
    f32 = jnp.float32
    r0 = x_ref[pl.ds(0, 7), :, :].reshape(7 * nb, 4 * XL)
    r1 = x_ref[pl.ds(1, 7), :, :].reshape(7 * nb, 4 * XL)
    c1 = []
    for p in range(4):
        lpp = jnp.concatenate([r0[:, p * XL:], r1[:, :(p + 1) * XL]], axis=1)
        c1.append(jnp.dot(lpp, B1_ref[...], preferred_element_type=f32))

    b1v = b1_ref[...]
    ye = jnp.maximum(c1[0], c1[1])
    ye = jnp.maximum(ye[:, :L1], ye[:, L1:])
    ye = jnp.maximum(ye + b1v, 0.0).astype(jnp.bfloat16)
    yo = jnp.maximum(c1[2], c1[3])
    yo = jnp.maximum(yo[:, :L1], yo[:, L1:])
    yo = jnp.maximum(yo + b1v, 0.0).astype(jnp.bfloat16)

    m = 6 * nb
    ye0 = jax.lax.slice(ye, (0, 0), (m, L1))
    ye1 = jax.lax.slice(ye, (nb, 0), (nb + m, L1))
    yo0 = jax.lax.slice(yo, (0, 0), (m, L1))
    yo1 = jax.lax.slice(yo, (nb, 0), (nb + m, L1))
    le = jnp.concatenate([ye0, yo0, ye1], axis=1)
    lo = jnp.concatenate([yo0, ye1, yo1], axis=1)
    l2 = jnp.concatenate([le, lo], axis=0)
    c2 = jnp.dot(l2, B2_ref[...], preferred_element_type=f32)

    y2 = jnp.maximum(c2[:m, :], c2[m:, :])
    y2 = jnp.maximum(y2[:, :L2], y2[:, L2:])
    y2 = jnp.maximum(y2 + b2_ref[...], 0.0)

    feats = jnp.concatenate(
        [jax.lax.slice(y2, (h * nb, 0), ((h + 1) * nb, L2))
         for h in range(P2)], axis=1).astype(jnp.bfloat16)
    h1 = jnp.dot(feats, fw1_ref[...], preferred_element_type=f32)
    h1 = jnp.maximum(h1 + fb1_ref[...], 0.0).astype(jnp.bfloat16)
    z = jnp.dot(h1, fw2_ref[...], preferred_element_type=f32)
    o_ref[...] = jnp.maximum(z + fb2_ref[...], 0.0)[:, :NOUT]


def _toeplitz_selector(size_in, size_out, k):
    w = jnp.arange(size_in)[:, None, None]
    order = jnp.concatenate(
        [jnp.arange(0, size_out, 2), jnp.arange(1, size_out, 2)])
    j = jnp.arange(k)[None, :, None]
    return (w == order[None, None, :] + j).astype(jnp.float32)


def _build_tables(w1, w2):
    w1r = w1.reshape(K1, K1, CIN, C1)
    e1 = _toeplitz_selector(H, O1, K1)
    b1 = jnp.einsum('wjv,ijco->icwvo', e1, w1r)
    b1 = b1.reshape(K1 * XL, 2, P1 * C1)
    b1 = jnp.pad(b1, ((0, 0), (0, 0), (0, L1 - P1 * C1)))
    B1 = b1.reshape(K1 * XL, S1)

    w2r = w2.reshape(K2, K2, C1, C2)
    e2 = _toeplitz_selector(P1, O2, K2)
    b2 = jnp.einsum('wjv,ijco->iwcvo', e2, w2r)
    b2 = b2.reshape(K2, P1 * C1, 2, P2 * C2)
    b2 = jnp.pad(b2, ((0, 0), (0, L1 - P1 * C1), (0, 0),
                      (0, L2 - P2 * C2)))
    B2 = b2.reshape(K2 * L1, S2)
    return B1.astype(jnp.bfloat16), B2.astype(jnp.bfloat16)


def kernel(x_nchw, w1, b1, w2, b2, fc1_w, fc1_b, fc2_w, fc2_b):
    n = x_nchw.shape[0]
    nb = min(NB, n)
    n_pad = ((n + nb - 1) // nb) * nb

    x = x_nchw.astype(jnp.bfloat16).reshape(n, CIN, 8, 4, H)
    x = jnp.transpose(x, (2, 0, 3, 1, 4)).reshape(8, n, 4 * XL)
    if n_pad > n:
        x = jnp.pad(x, ((0, 0), (0, n_pad - n), (0, 0)))

    B1, B2 = _build_tables(w1, w2)
    b1t = jnp.pad(jnp.tile(b1, (1, P1)), ((0, 0), (0, L1 - P1 * C1)))
    b2t = jnp.pad(jnp.tile(b2, (1, P2)), ((0, 0), (0, L2 - P2 * C2)))
    fw1 = fc1_w.reshape(P2, P2 * C2, FC1)
    fw1 = jnp.pad(fw1, ((0, 0), (0, L2 - P2 * C2), (0, 0)))
    fw1 = fw1.reshape(P2 * L2, FC1).astype(jnp.bfloat16)
    fw2 = fc2_w.astype(jnp.bfloat16)

    out = pl.pallas_call(
        functools.partial(_net_kernel, nb=nb),
        out_shape=jax.ShapeDtypeStruct((n_pad, NOUT), jnp.float32),
        grid=(n_pad // nb,),
        in_specs=[
            pl.BlockSpec((8, nb, 4 * XL), lambda i: (0, i, 0)),
            pl.BlockSpec((K1 * XL, S1), lambda i: (0, 0)),
            pl.BlockSpec((1, L1), lambda i: (0, 0)),
            pl.BlockSpec((K2 * L1, S2), lambda i: (0, 0)),
            pl.BlockSpec((1, L2), lambda i: (0, 0)),
            pl.BlockSpec((P2 * L2, FC1), lambda i: (0, 0)),
            pl.BlockSpec((1, FC1), lambda i: (0, 0)),
            pl.BlockSpec((FC1, LANES), lambda i: (0, 0)),
            pl.BlockSpec((1, LANES), lambda i: (0, 0)),
        ],
        out_specs=pl.BlockSpec((nb, NOUT), lambda i: (i, 0)),
        compiler_params=pltpu.CompilerParams(
            dimension_semantics=("parallel",)),
    )(x, B1, b1t, B2, b2t, fw1, fc1_b, fw2, fc2_b)
    return out[:n] if n_pad > n else out
